# baseline (speedup 1.0000x reference)
import functools

import jax
import jax.numpy as jnp
from jax import lax
from jax.experimental import pallas as pl
from jax.experimental.pallas import tpu as pltpu
from jax.experimental.pallas import tpu_sc as plsc

_N = 10000
_E = 320000
_D = 128

_NC = 2
_NS = 16
_NW = _NC * _NS
_EW = _E // _NW

_GC = 80
_GSUB = 5
_GBUF = _GC * _GSUB
_GITER = _EW // _GBUF

_SCC = 80
_SCH = _EW // _SCC
_NP = 10240
_NPT = _NP // _NS

_sc_mesh = plsc.VectorSubcoreMesh(
    core_axis_name="c", subcore_axis_name="s",
    num_cores=_NC, num_subcores=_NS)


@functools.partial(
    pl.kernel,
    out_type=(jax.ShapeDtypeStruct((_E, _D), jnp.float32),
              jax.ShapeDtypeStruct((_E, _D), jnp.float32)),
    mesh=_sc_mesh,
    scratch_types=[
        pltpu.VMEM((_EW,), jnp.int32),
        pltpu.VMEM((_EW,), jnp.int32),
        pltpu.VMEM((_GBUF, _D), jnp.float32),
        pltpu.VMEM((_GBUF, _D), jnp.float32),
        pltpu.SemaphoreType.DMA,
    ],
)
def _sc_gather(p1_hbm, p2_hbm, dst_hbm, src_hbm, g1_hbm, g2_hbm,
               dst_v, src_v, buf1, buf2, sem):
    wid = lax.axis_index("s") * _NC + lax.axis_index("c")
    base = pl.multiple_of(wid * _EW, _EW)
    pltpu.sync_copy(dst_hbm.at[pl.ds(base, _EW)], dst_v)
    pltpu.sync_copy(src_hbm.at[pl.ds(base, _EW)], src_v)

    def step(i, carry):
        off = pl.multiple_of(i * _GBUF, _GBUF)
        cps = []
        for j in range(_GSUB):
            cps.append(pltpu.async_copy(
                p1_hbm.at[dst_v.at[pl.ds(off + j * _GC, _GC)]],
                buf1.at[pl.ds(j * _GC, _GC)], sem))
        for j in range(_GSUB):
            cps.append(pltpu.async_copy(
                p2_hbm.at[src_v.at[pl.ds(off + j * _GC, _GC)]],
                buf2.at[pl.ds(j * _GC, _GC)], sem))
        for c in cps:
            c.wait()
        pltpu.sync_copy(buf1, g1_hbm.at[pl.ds(base + off, _GBUF)])
        pltpu.sync_copy(buf2, g2_hbm.at[pl.ds(base + off, _GBUF)])
        return carry

    lax.fori_loop(0, _GITER, step, 0)


@functools.partial(
    pl.kernel,
    out_type=jax.ShapeDtypeStruct((_NC, _NP, _D), jnp.float32),
    mesh=_sc_mesh,
    scratch_types=[
        pltpu.VMEM_SHARED((_NP, _D), jnp.float32),
        pltpu.VMEM((_SCH, _SCC), jnp.int32),
        pltpu.VMEM((_SCC, _D), jnp.float32),
        pltpu.VMEM((_SCC, _D), jnp.float32),
        pltpu.SemaphoreType.DMA,
    ],
)
def _sc_scatter(msg_hbm, dst3_hbm, zero_hbm, out_hbm, accum, dst_v, m0, m1, sem):
    cid = lax.axis_index("c")
    sid = lax.axis_index("s")
    wid = sid * _NC + cid
    pltpu.sync_copy(zero_hbm.at[pl.ds(sid * _NPT, _NPT)],
                    accum.at[pl.ds(sid * _NPT, _NPT)])
    pltpu.sync_copy(dst3_hbm.at[wid], dst_v)
    plsc.subcore_barrier()

    base = pl.multiple_of(wid * _EW, _EW)

    def load(b, buf):
        pltpu.async_copy(msg_hbm.at[pl.ds(base + b * _SCC, _SCC)], buf, sem)

    def wait_load(buf):
        pltpu.make_async_copy(msg_hbm.at[pl.ds(0, _SCC)], buf, sem).wait()

    load(0, m0)

    def step(k, carry):
        b = k * 2
        wait_load(m0)
        load(b + 1, m1)
        pltpu.sync_copy(m0, accum.at[dst_v.at[b]], add=True)
        wait_load(m1)
        load(b + 2, m0)
        pltpu.sync_copy(m1, accum.at[dst_v.at[b + 1]], add=True)
        return carry

    lax.fori_loop(0, (_SCH - 1) // 2, step, 0)
    wait_load(m0)
    pltpu.sync_copy(m0, accum.at[dst_v.at[_SCH - 1]], add=True)
    plsc.subcore_barrier()
    pltpu.sync_copy(accum.at[pl.ds(sid * _NPT, _NPT)],
                    out_hbm.at[cid].at[pl.ds(sid * _NPT, _NPT)])


def _silu(v):
    return v * jax.nn.sigmoid(v)


def _tc_node_pre(x, wab):
    def body(x_ref, w_ref, p1_ref, p2_ref):
        p12 = jnp.dot(x_ref[...], w_ref[...],
                      preferred_element_type=jnp.float32)
        p1_ref[...] = p12[:, :_D]
        p2_ref[...] = p12[:, _D:]

    return pl.pallas_call(
        body,
        out_shape=(jax.ShapeDtypeStruct((_N, _D), jnp.float32),
                   jax.ShapeDtypeStruct((_N, _D), jnp.float32)),
    )(x, wab)


_BE = 4000


def _tc_edge(ea, g1, g2, we1, wag, wm2, be1, cag, bm2):
    def body(ea_ref, g1_ref, g2_ref, we1_ref, wag_ref, wm2_ref,
             be1_ref, cag_ref, bm2_ref, msg_ref):
        t = jnp.dot(ea_ref[...], we1_ref[...],
                    preferred_element_type=jnp.float32) + be1_ref[...]
        t = _silu(t)
        tt = jnp.dot(t, wag_ref[...],
                     preferred_element_type=jnp.float32) + cag_ref[...]
        pre = g1_ref[...] + g2_ref[...] + tt[:, :_D]
        gate = jax.nn.sigmoid(tt[:, _D:])
        p = _silu(pre)
        msg_ref[...] = (jnp.dot(p, wm2_ref[...],
                                preferred_element_type=jnp.float32)
                        + bm2_ref[...]) * gate

    grid = _E // _BE
    return pl.pallas_call(
        body,
        grid=(grid,),
        in_specs=[
            pl.BlockSpec((_BE, 32), lambda i: (i, 0)),
            pl.BlockSpec((_BE, _D), lambda i: (i, 0)),
            pl.BlockSpec((_BE, _D), lambda i: (i, 0)),
            pl.BlockSpec((32, _D), lambda i: (0, 0)),
            pl.BlockSpec((_D, 2 * _D), lambda i: (0, 0)),
            pl.BlockSpec((_D, _D), lambda i: (0, 0)),
            pl.BlockSpec((1, _D), lambda i: (0, 0)),
            pl.BlockSpec((1, 2 * _D), lambda i: (0, 0)),
            pl.BlockSpec((1, _D), lambda i: (0, 0)),
        ],
        out_specs=pl.BlockSpec((_BE, _D), lambda i: (i, 0)),
        out_shape=jax.ShapeDtypeStruct((_E, _D), jnp.float32),
    )(ea, g1, g2, we1, wag, wm2, be1, cag, bm2)


def _tc_update(p0, p1, x, wu1, wu2, bu1, bu2, gamma, beta):
    def body(p0_ref, p1_ref, x_ref, wu1_ref, wu2_ref,
             bu1_ref, bu2_ref, g_ref, b_ref, o_ref):
        xv = x_ref[...]
        aggr = p0_ref[...][:_N] + p1_ref[...][:_N]
        cat = jnp.concatenate([aggr, xv], axis=1)
        u = jnp.dot(cat, wu1_ref[...],
                    preferred_element_type=jnp.float32) + bu1_ref[...]
        u = _silu(u)
        h = xv + jnp.dot(u, wu2_ref[...],
                         preferred_element_type=jnp.float32) + bu2_ref[...]
        mu = jnp.mean(h, axis=1, keepdims=True)
        hc = h - mu
        var = jnp.mean(hc * hc, axis=1, keepdims=True)
        o_ref[...] = hc * lax.rsqrt(var + 1e-5) * g_ref[...] + b_ref[...]

    return pl.pallas_call(
        body,
        out_shape=jax.ShapeDtypeStruct((_N, _D), jnp.float32),
    )(p0, p1, x, wu1, wu2, bu1, bu2, gamma, beta)


def kernel(x, edge_index, edge_attr, We1, be1, We2, be2, Wm1, bm1, Wm2, bm2,
           Wu1, bu1, Wu2, bu2, Wg, bg, ln_gamma, ln_beta):
    src = edge_index[0]
    dst = edge_index[1]

    wm1a = Wm1[:_D]
    wm1b = Wm1[_D:2 * _D]
    wm1c = Wm1[2 * _D:]
    wa = We2 @ wm1c
    wg2 = We2 @ Wg
    ca = be2 @ wm1c + bm1
    cg = be2 @ Wg + bg
    wab = jnp.concatenate([wm1a, wm1b], axis=1)
    wag = jnp.concatenate([wa, wg2], axis=1)
    cag = jnp.concatenate([ca, cg], axis=0).reshape(1, 2 * _D)

    p1, p2 = _tc_node_pre(x, wab)
    g1, g2 = _sc_gather(p1, p2, dst, src)
    msg = _tc_edge(edge_attr, g1, g2, We1, wag, Wm2,
                   be1.reshape(1, _D), cag, bm2.reshape(1, _D))
    dst3 = dst.reshape(_NW, _SCH, _SCC)
    zeros = jnp.zeros((_NP, _D), jnp.float32)
    parts = _sc_scatter(msg, dst3, zeros)
    out = _tc_update(parts[0], parts[1], x, Wu1, Wu2,
                     bu1.reshape(1, _D), bu2.reshape(1, _D),
                     ln_gamma.reshape(1, _D), ln_beta.reshape(1, _D))
    return out


# speedup vs baseline: 1.0764x; 1.0764x over previous
import functools

import jax
import jax.numpy as jnp
from jax import lax
from jax.experimental import pallas as pl
from jax.experimental.pallas import tpu as pltpu
from jax.experimental.pallas import tpu_sc as plsc

_N = 10000
_E = 320000
_D = 128

_NC = 2
_NS = 16
_NW = _NC * _NS
_EW = _E // _NW

_GC = 40
_GSUB = 5
_GBUF = _GC * _GSUB
_GITER = _EW // _GBUF
_GPAIR = _GITER // 2

_SCC = 80
_SCH = _EW // _SCC
_NP = 10240
_NPT = _NP // _NS

_sc_mesh = plsc.VectorSubcoreMesh(
    core_axis_name="c", subcore_axis_name="s",
    num_cores=_NC, num_subcores=_NS)


@functools.partial(
    pl.kernel,
    out_type=(jax.ShapeDtypeStruct((_E, _D), jnp.float32),
              jax.ShapeDtypeStruct((_E, _D), jnp.float32)),
    mesh=_sc_mesh,
    scratch_types=[
        pltpu.VMEM((_EW,), jnp.int32),
        pltpu.VMEM((_EW,), jnp.int32),
        pltpu.VMEM((_GBUF, _D), jnp.float32),
        pltpu.VMEM((_GBUF, _D), jnp.float32),
        pltpu.VMEM((_GBUF, _D), jnp.float32),
        pltpu.VMEM((_GBUF, _D), jnp.float32),
        pltpu.SemaphoreType.DMA,
        pltpu.SemaphoreType.DMA,
        pltpu.SemaphoreType.DMA,
    ],
)
def _sc_gather(p1_hbm, p2_hbm, dst_hbm, src_hbm, g1_hbm, g2_hbm,
               dst_v, src_v, b1a, b2a, b1b, b2b, semg, semwa, semwb):
    wid = lax.axis_index("s") * _NC + lax.axis_index("c")
    base = pl.multiple_of(wid * _EW, _EW)
    pltpu.sync_copy(dst_hbm.at[pl.ds(base, _EW)], dst_v)
    pltpu.sync_copy(src_hbm.at[pl.ds(base, _EW)], src_v)

    def fire(i, b1, b2):
        off = pl.multiple_of(i * _GBUF, _GBUF)
        for j in range(_GSUB):
            pltpu.async_copy(
                p1_hbm.at[dst_v.at[pl.ds(off + j * _GC, _GC)]],
                b1.at[pl.ds(j * _GC, _GC)], semg)
            pltpu.async_copy(
                p2_hbm.at[src_v.at[pl.ds(off + j * _GC, _GC)]],
                b2.at[pl.ds(j * _GC, _GC)], semg)

    def drain(b1, b2):
        for j in range(_GSUB):
            pltpu.make_async_copy(
                p1_hbm.at[pl.ds(0, _GC)], b1.at[pl.ds(j * _GC, _GC)],
                semg).wait()
            pltpu.make_async_copy(
                p2_hbm.at[pl.ds(0, _GC)], b2.at[pl.ds(j * _GC, _GC)],
                semg).wait()

    def write(i, b1, b2, semw):
        off = pl.multiple_of(i * _GBUF, _GBUF)
        pltpu.async_copy(b1, g1_hbm.at[pl.ds(base + off, _GBUF)], semw)
        pltpu.async_copy(b2, g2_hbm.at[pl.ds(base + off, _GBUF)], semw)

    def wait_write(b1, b2, semw):
        pltpu.make_async_copy(b1, g1_hbm.at[pl.ds(base, _GBUF)], semw).wait()
        pltpu.make_async_copy(b2, g2_hbm.at[pl.ds(base, _GBUF)], semw).wait()

    fire(0, b1a, b2a)

    def step(k, carry):
        b = k * 2
        drain(b1a, b2a)

        @pl.when(k > 0)
        def _():
            wait_write(b1b, b2b, semwb)

        fire(b + 1, b1b, b2b)
        write(b, b1a, b2a, semwa)
        drain(b1b, b2b)
        wait_write(b1a, b2a, semwa)

        @pl.when(k < _GPAIR - 1)
        def _():
            fire(b + 2, b1a, b2a)

        write(b + 1, b1b, b2b, semwb)
        return carry

    lax.fori_loop(0, _GPAIR, step, 0)
    wait_write(b1b, b2b, semwb)


@functools.partial(
    pl.kernel,
    out_type=jax.ShapeDtypeStruct((_NC, _NP, _D), jnp.float32),
    mesh=_sc_mesh,
    scratch_types=[
        pltpu.VMEM_SHARED((_NP, _D), jnp.float32),
        pltpu.VMEM((_SCH, _SCC), jnp.int32),
        pltpu.VMEM((_SCC, _D), jnp.float32),
        pltpu.VMEM((_SCC, _D), jnp.float32),
        pltpu.SemaphoreType.DMA,
    ],
)
def _sc_scatter(msg_hbm, dst3_hbm, zero_hbm, out_hbm, accum, dst_v, m0, m1, sem):
    cid = lax.axis_index("c")
    sid = lax.axis_index("s")
    wid = sid * _NC + cid
    pltpu.sync_copy(zero_hbm.at[pl.ds(sid * _NPT, _NPT)],
                    accum.at[pl.ds(sid * _NPT, _NPT)])
    pltpu.sync_copy(dst3_hbm.at[wid], dst_v)
    plsc.subcore_barrier()

    base = pl.multiple_of(wid * _EW, _EW)

    def load(b, buf):
        pltpu.async_copy(msg_hbm.at[pl.ds(base + b * _SCC, _SCC)], buf, sem)

    def wait_load(buf):
        pltpu.make_async_copy(msg_hbm.at[pl.ds(0, _SCC)], buf, sem).wait()

    load(0, m0)

    def step(k, carry):
        b = k * 2
        wait_load(m0)
        load(b + 1, m1)
        pltpu.sync_copy(m0, accum.at[dst_v.at[b]], add=True)
        wait_load(m1)
        load(b + 2, m0)
        pltpu.sync_copy(m1, accum.at[dst_v.at[b + 1]], add=True)
        return carry

    lax.fori_loop(0, (_SCH - 1) // 2, step, 0)
    wait_load(m0)
    pltpu.sync_copy(m0, accum.at[dst_v.at[_SCH - 1]], add=True)
    plsc.subcore_barrier()
    pltpu.sync_copy(accum.at[pl.ds(sid * _NPT, _NPT)],
                    out_hbm.at[cid].at[pl.ds(sid * _NPT, _NPT)])


def _silu(v):
    return v * jax.nn.sigmoid(v)


def _tc_node_pre(x, wab):
    def body(x_ref, w_ref, p1_ref, p2_ref):
        p12 = jnp.dot(x_ref[...], w_ref[...],
                      preferred_element_type=jnp.float32)
        p1_ref[...] = p12[:, :_D]
        p2_ref[...] = p12[:, _D:]

    return pl.pallas_call(
        body,
        out_shape=(jax.ShapeDtypeStruct((_N, _D), jnp.float32),
                   jax.ShapeDtypeStruct((_N, _D), jnp.float32)),
    )(x, wab)


_BE = 4000


def _tc_edge(ea, g1, g2, we1, wag, wm2, be1, cag, bm2):
    def body(ea_ref, g1_ref, g2_ref, we1_ref, wag_ref, wm2_ref,
             be1_ref, cag_ref, bm2_ref, msg_ref):
        t = jnp.dot(ea_ref[...], we1_ref[...],
                    preferred_element_type=jnp.float32) + be1_ref[...]
        t = _silu(t)
        tt = jnp.dot(t, wag_ref[...],
                     preferred_element_type=jnp.float32) + cag_ref[...]
        pre = g1_ref[...] + g2_ref[...] + tt[:, :_D]
        gate = jax.nn.sigmoid(tt[:, _D:])
        p = _silu(pre)
        msg_ref[...] = (jnp.dot(p, wm2_ref[...],
                                preferred_element_type=jnp.float32)
                        + bm2_ref[...]) * gate

    grid = _E // _BE
    return pl.pallas_call(
        body,
        grid=(grid,),
        in_specs=[
            pl.BlockSpec((_BE, 32), lambda i: (i, 0)),
            pl.BlockSpec((_BE, _D), lambda i: (i, 0)),
            pl.BlockSpec((_BE, _D), lambda i: (i, 0)),
            pl.BlockSpec((32, _D), lambda i: (0, 0)),
            pl.BlockSpec((_D, 2 * _D), lambda i: (0, 0)),
            pl.BlockSpec((_D, _D), lambda i: (0, 0)),
            pl.BlockSpec((1, _D), lambda i: (0, 0)),
            pl.BlockSpec((1, 2 * _D), lambda i: (0, 0)),
            pl.BlockSpec((1, _D), lambda i: (0, 0)),
        ],
        out_specs=pl.BlockSpec((_BE, _D), lambda i: (i, 0)),
        out_shape=jax.ShapeDtypeStruct((_E, _D), jnp.float32),
    )(ea, g1, g2, we1, wag, wm2, be1, cag, bm2)


def _tc_update(p0, p1, x, wu1, wu2, bu1, bu2, gamma, beta):
    def body(p0_ref, p1_ref, x_ref, wu1_ref, wu2_ref,
             bu1_ref, bu2_ref, g_ref, b_ref, o_ref):
        xv = x_ref[...]
        aggr = p0_ref[...][:_N] + p1_ref[...][:_N]
        cat = jnp.concatenate([aggr, xv], axis=1)
        u = jnp.dot(cat, wu1_ref[...],
                    preferred_element_type=jnp.float32) + bu1_ref[...]
        u = _silu(u)
        h = xv + jnp.dot(u, wu2_ref[...],
                         preferred_element_type=jnp.float32) + bu2_ref[...]
        mu = jnp.mean(h, axis=1, keepdims=True)
        hc = h - mu
        var = jnp.mean(hc * hc, axis=1, keepdims=True)
        o_ref[...] = hc * lax.rsqrt(var + 1e-5) * g_ref[...] + b_ref[...]

    return pl.pallas_call(
        body,
        out_shape=jax.ShapeDtypeStruct((_N, _D), jnp.float32),
    )(p0, p1, x, wu1, wu2, bu1, bu2, gamma, beta)


def kernel(x, edge_index, edge_attr, We1, be1, We2, be2, Wm1, bm1, Wm2, bm2,
           Wu1, bu1, Wu2, bu2, Wg, bg, ln_gamma, ln_beta):
    src = edge_index[0]
    dst = edge_index[1]

    wm1a = Wm1[:_D]
    wm1b = Wm1[_D:2 * _D]
    wm1c = Wm1[2 * _D:]
    wa = We2 @ wm1c
    wg2 = We2 @ Wg
    ca = be2 @ wm1c + bm1
    cg = be2 @ Wg + bg
    wab = jnp.concatenate([wm1a, wm1b], axis=1)
    wag = jnp.concatenate([wa, wg2], axis=1)
    cag = jnp.concatenate([ca, cg], axis=0).reshape(1, 2 * _D)

    p1, p2 = _tc_node_pre(x, wab)
    g1, g2 = _sc_gather(p1, p2, dst, src)
    msg = _tc_edge(edge_attr, g1, g2, We1, wag, Wm2,
                   be1.reshape(1, _D), cag, bm2.reshape(1, _D))
    dst3 = dst.reshape(_NW, _SCH, _SCC)
    zeros = jnp.zeros((_NP, _D), jnp.float32)
    parts = _sc_scatter(msg, dst3, zeros)
    out = _tc_update(parts[0], parts[1], x, Wu1, Wu2,
                     bu1.reshape(1, _D), bu2.reshape(1, _D),
                     ln_gamma.reshape(1, _D), ln_beta.reshape(1, _D))
    return out
